# Initial kernel scaffold; baseline (speedup 1.0000x reference)
import jax
import jax.numpy as jnp
from jax.experimental import pallas as pl


def kernel(gold_encoding_inds, latents, epc, W):
    raise NotImplementedError("write your pallas kernel here")


# speedup vs baseline: 10.9257x; 10.9257x over previous
import functools

import jax
import jax.numpy as jnp
from jax import lax
from jax.experimental import pallas as pl
from jax.experimental.pallas import tpu as pltpu
from jax.experimental.pallas import tpu_sc as plsc

K = 8192
D = 256
BETA = 0.25
N = 8192
NC, NS, L = 2, 16, 16
NW = NC * NS
B_PER_W = N // NW
CHUNK = 128
NCHUNK = B_PER_W // CHUNK

_mesh = plsc.VectorSubcoreMesh(
    core_axis_name="c", subcore_axis_name="s", num_cores=NC, num_subcores=NS)


@functools.partial(
    pl.kernel,
    out_type=(
        jax.ShapeDtypeStruct((N, D), jnp.float32),
        jax.ShapeDtypeStruct((N,), jnp.float32),
    ),
    mesh=_mesh,
    scratch_types=[
        pltpu.VMEM((B_PER_W,), jnp.int32),
        pltpu.VMEM((CHUNK, D), jnp.float32),
        pltpu.VMEM((CHUNK, D), jnp.float32),
        pltpu.VMEM((B_PER_W,), jnp.float32),
        pltpu.SemaphoreType.DMA,
        pltpu.SemaphoreType.DMA,
    ],
    compiler_params=pltpu.CompilerParams(
        use_tc_tiling_on_sc=False, needs_layout_passes=False),
)
def _vq_sc(idx_hbm, lat_hbm, w_hbm, q_hbm, loss_hbm,
           idx_v, rows_v, lat_v, loss_v, sem_g, sem_l):
    wid = lax.axis_index("s") * NC + lax.axis_index("c")
    base = wid * B_PER_W
    pltpu.sync_copy(idx_hbm.at[pl.ds(base, B_PER_W)], idx_v)
    scale = jnp.float32((1.0 + BETA) / D)
    for c in range(NCHUNK):
        off = base + c * CHUNK
        gath = pltpu.async_copy(
            w_hbm.at[idx_v.at[pl.ds(c * CHUNK, CHUNK)]], rows_v, sem_g)
        latc = pltpu.async_copy(lat_hbm.at[pl.ds(off, CHUNK)], lat_v, sem_l)
        gath.wait()
        latc.wait()

        lane = lax.iota(jnp.int32, L)

        def group_body(g, carry):
            tok = g * L + lane

            def d_body(dd, accs):
                a0, a1 = accs
                d0 = jnp.full((L,), 2 * dd, jnp.int32)
                d1 = jnp.full((L,), 2 * dd + 1, jnp.int32)
                r0 = plsc.load_gather(rows_v, [tok, d0])
                x0 = plsc.load_gather(lat_v, [tok, d0])
                r1 = plsc.load_gather(rows_v, [tok, d1])
                x1 = plsc.load_gather(lat_v, [tok, d1])
                e0 = r0 - x0
                e1 = r1 - x1
                return a0 + e0 * e0, a1 + e1 * e1

            acc0, acc1 = lax.fori_loop(
                0, D // 2, d_body,
                (jnp.zeros((L,), jnp.float32), jnp.zeros((L,), jnp.float32)),
                unroll=4)
            loss_v[pl.ds(c * CHUNK + g * L, L)] = (acc0 + acc1) * scale
            return carry

        lax.fori_loop(0, CHUNK // L, group_body, 0)
        pltpu.sync_copy(rows_v, q_hbm.at[pl.ds(off, CHUNK)])
    pltpu.sync_copy(loss_v, loss_hbm.at[pl.ds(base, B_PER_W)])


def kernel(gold_encoding_inds, latents, epc, W):
    bs, t, d = latents.shape
    flat_latents = latents.reshape(bs * t, d)
    idx = gold_encoding_inds[:, 0].astype(jnp.int32)
    q, loss = _vq_sc(idx, flat_latents, W)
    gold_quantized = q.reshape(latents.shape)
    vq_loss = loss.reshape(bs, t)
    return gold_quantized, vq_loss, gold_encoding_inds.T


# speedup vs baseline: 17.1000x; 1.5651x over previous
import functools

import jax
import jax.numpy as jnp
from jax import lax
from jax.experimental import pallas as pl
from jax.experimental.pallas import tpu as pltpu
from jax.experimental.pallas import tpu_sc as plsc

K = 8192
D = 256
BETA = 0.25
N = 8192
NC, NS, L = 2, 16, 16
NW = NC * NS
B_PER_W = N // NW
CHUNK = 128
NCHUNK = B_PER_W // CHUNK

_mesh = plsc.VectorSubcoreMesh(
    core_axis_name="c", subcore_axis_name="s", num_cores=NC, num_subcores=NS)


@functools.partial(
    pl.kernel,
    out_type=(
        jax.ShapeDtypeStruct((N, D), jnp.float32),
        jax.ShapeDtypeStruct((N,), jnp.float32),
    ),
    mesh=_mesh,
    scratch_types=[
        pltpu.VMEM((B_PER_W,), jnp.int32),
        pltpu.VMEM((CHUNK, D), jnp.float32),
        pltpu.VMEM((CHUNK, D), jnp.float32),
        pltpu.VMEM((B_PER_W,), jnp.float32),
        pltpu.SemaphoreType.DMA,
        pltpu.SemaphoreType.DMA,
        pltpu.SemaphoreType.DMA,
    ],
    compiler_params=pltpu.CompilerParams(
        use_tc_tiling_on_sc=False, needs_layout_passes=False),
)
def _vq_sc(idx_hbm, lat_hbm, w_hbm, q_hbm, loss_hbm,
           idx_v, rows_v, lat_v, loss_v, sem_g, sem_l, sem_o):
    wid = lax.axis_index("s") * NC + lax.axis_index("c")
    base = wid * B_PER_W
    pltpu.sync_copy(idx_hbm.at[pl.ds(base, B_PER_W)], idx_v)
    scale = jnp.float32((1.0 + BETA) / D)
    lane = lax.iota(jnp.int32, L)
    masks = [(lane & d) == 0 for d in (1, 2, 4, 8)]
    for c in range(NCHUNK):
        off = base + c * CHUNK
        gath = pltpu.async_copy(
            w_hbm.at[idx_v.at[pl.ds(c * CHUNK, CHUNK)]], rows_v, sem_g)
        latc = pltpu.async_copy(lat_hbm.at[pl.ds(off, CHUNK)], lat_v, sem_l)
        gath.wait()
        latc.wait()
        outc = pltpu.async_copy(rows_v, q_hbm.at[pl.ds(off, CHUNK)], sem_o)

        def group_body(g, carry):
            accs = []
            for u in range(L):
                t = g * L + u
                a0 = jnp.zeros((L,), jnp.float32)
                a1 = jnp.zeros((L,), jnp.float32)
                for j in range(D // L):
                    e = rows_v[t, pl.ds(j * L, L)] - lat_v[t, pl.ds(j * L, L)]
                    if j % 2 == 0:
                        a0 = a0 + e * e
                    else:
                        a1 = a1 + e * e
                accs.append(a0 + a1)
            for si, dist in enumerate((1, 2, 4, 8)):
                nxt = []
                for p in range(0, len(accs), 2):
                    a, b = accs[p], accs[p + 1]
                    pa = a.at[lane ^ dist].get(mode="promise_in_bounds")
                    pb = b.at[lane ^ dist].get(mode="promise_in_bounds")
                    nxt.append(jnp.where(masks[si], a + pa, b + pb))
                accs = nxt
            loss_v[pl.ds(c * CHUNK + g * L, L)] = accs[0] * scale
            return carry

        lax.fori_loop(0, CHUNK // L, group_body, 0)
        outc.wait()
    pltpu.sync_copy(loss_v, loss_hbm.at[pl.ds(base, B_PER_W)])


def kernel(gold_encoding_inds, latents, epc, W):
    bs, t, d = latents.shape
    flat_latents = latents.reshape(bs * t, d)
    idx = gold_encoding_inds[:, 0].astype(jnp.int32)
    q, loss = _vq_sc(idx, flat_latents, W)
    gold_quantized = q.reshape(latents.shape)
    vq_loss = loss.reshape(bs, t)
    return gold_quantized, vq_loss, gold_encoding_inds.T


# speedup vs baseline: 28.7369x; 1.6805x over previous
import functools

import jax
import jax.numpy as jnp
from jax import lax
from jax.experimental import pallas as pl
from jax.experimental.pallas import tpu as pltpu
from jax.experimental.pallas import tpu_sc as plsc

K = 8192
D = 256
BETA = 0.25
N = 8192
NC, NS, L = 2, 16, 16
NW = NC * NS
B_PER_W = N // NW
CHUNK = 64
NCHUNK = B_PER_W // CHUNK
HR = 2 * CHUNK

_mesh = plsc.VectorSubcoreMesh(
    core_axis_name="c", subcore_axis_name="s", num_cores=NC, num_subcores=NS)


@functools.partial(
    pl.kernel,
    out_type=(
        jax.ShapeDtypeStruct((2 * N, 128), jnp.float32),
        jax.ShapeDtypeStruct((N,), jnp.float32),
    ),
    mesh=_mesh,
    scratch_types=[
        pltpu.VMEM((B_PER_W,), jnp.int32),
        pltpu.VMEM((2 * B_PER_W,), jnp.int32),
        pltpu.VMEM((2, HR, 128), jnp.float32),
        pltpu.VMEM((2, HR, 128), jnp.float32),
        pltpu.VMEM((B_PER_W,), jnp.float32),
        pltpu.SemaphoreType.DMA,
        pltpu.SemaphoreType.DMA,
        pltpu.SemaphoreType.DMA,
        pltpu.SemaphoreType.DMA,
        pltpu.SemaphoreType.DMA,
        pltpu.SemaphoreType.DMA,
    ],
    compiler_params=pltpu.CompilerParams(
        use_tc_tiling_on_sc=False, needs_layout_passes=False),
)
def _vq_sc(idx_hbm, lat_hbm, w_hbm, q_hbm, loss_hbm,
           idx_v, idx2_v, rows2, lat2, loss_v,
           sg0, sg1, sl0, sl1, so0, so1):
    wid = lax.axis_index("s") * NC + lax.axis_index("c")
    base = wid * B_PER_W
    rbase = 2 * base
    pltpu.sync_copy(idx_hbm.at[pl.ds(base, B_PER_W)], idx_v)
    scale = jnp.float32((1.0 + BETA) / D)
    lane = lax.iota(jnp.int32, L)
    masks = [(lane & d) == 0 for d in (1, 2, 4, 8)]

    def expand(g16, carry):
        k = idx_v[pl.ds(g16 * L, L)]
        i0 = ((k >> 3) << 4) | (k & 7)
        j = g16 * L + lane
        p0 = ((j >> 3) << 4) | (j & 7)
        plsc.store_scatter(idx2_v, [p0], i0)
        plsc.store_scatter(idx2_v, [p0 | 8], i0 | 8)
        return carry

    lax.fori_loop(0, B_PER_W // L, expand, 0)

    sgs = [sg0, sg1]
    sls = [sl0, sl1]
    sos = [so0, so1]

    def issue(c):
        b = c % 2
        g = pltpu.async_copy(
            w_hbm.at[idx2_v.at[pl.ds(c * HR, HR)]], rows2.at[b], sgs[b])
        l = pltpu.async_copy(
            lat_hbm.at[pl.ds(rbase + c * HR, HR)], lat2.at[b], sls[b])
        return g, l

    pend = issue(0)
    outs = [None, None]
    for c in range(NCHUNK):
        b = c % 2
        pend[0].wait()
        pend[1].wait()
        if c + 1 < NCHUNK:
            nb = (c + 1) % 2
            if outs[nb] is not None:
                outs[nb].wait()
                outs[nb] = None
            pend = issue(c + 1)
        outc = pltpu.async_copy(
            rows2.at[b], q_hbm.at[pl.ds(rbase + c * HR, HR)], sos[b])
        outs[b] = outc

        def group_body(g, carry):
            accs = []
            for u in range(L):
                a0 = jnp.zeros((L,), jnp.float32)
                a1 = jnp.zeros((L,), jnp.float32)
                for cc in range(2):
                    r = 32 * g + 16 * (u >> 3) + (u & 7) + 8 * cc
                    for j in range(128 // L):
                        e = (rows2[b, r, pl.ds(j * L, L)]
                             - lat2[b, r, pl.ds(j * L, L)])
                        if j % 2 == 0:
                            a0 = a0 + e * e
                        else:
                            a1 = a1 + e * e
                accs.append(a0 + a1)
            for si, dist in enumerate((1, 2, 4, 8)):
                nxt = []
                for p in range(0, len(accs), 2):
                    x, y = accs[p], accs[p + 1]
                    px = x.at[lane ^ dist].get(mode="promise_in_bounds")
                    py = y.at[lane ^ dist].get(mode="promise_in_bounds")
                    nxt.append(jnp.where(masks[si], x + px, y + py))
                accs = nxt
            loss_v[pl.ds(c * CHUNK + g * L, L)] = accs[0] * scale
            return carry

        lax.fori_loop(0, CHUNK // L, group_body, 0)
    for o in outs:
        if o is not None:
            o.wait()
    pltpu.sync_copy(loss_v, loss_hbm.at[pl.ds(base, B_PER_W)])


def kernel(gold_encoding_inds, latents, epc, W):
    bs, t, d = latents.shape
    n = bs * t
    idx = gold_encoding_inds[:, 0].astype(jnp.int32)
    w_r = W.reshape(K // 8, 8, 2, 128).transpose(0, 2, 1, 3).reshape(2 * K, 128)
    lat_r = latents.reshape(bs, t // 8, 8, 2, 128).transpose(0, 1, 3, 2, 4)
    lat_r = lat_r.reshape(2 * n, 128)
    q_r, loss = _vq_sc(idx, lat_r, w_r)
    gold_quantized = (q_r.reshape(bs, t // 8, 2, 8, 128)
                      .transpose(0, 1, 3, 2, 4).reshape(bs, t, d))
    vq_loss = loss.reshape(bs, t)
    return gold_quantized, vq_loss, gold_encoding_inds.T


# speedup vs baseline: 29.8224x; 1.0378x over previous
import functools

import jax
import jax.numpy as jnp
from jax import lax
from jax.experimental import pallas as pl
from jax.experimental.pallas import tpu as pltpu
from jax.experimental.pallas import tpu_sc as plsc

K = 8192
D = 256
BETA = 0.25
N = 8192
NC, NS, L = 2, 16, 16
NW = NC * NS
B_PER_W = N // NW
CHUNK = 64
NCHUNK = B_PER_W // CHUNK
HR = 2 * CHUNK

_mesh = plsc.VectorSubcoreMesh(
    core_axis_name="c", subcore_axis_name="s", num_cores=NC, num_subcores=NS)


@functools.partial(
    pl.kernel,
    out_type=(
        jax.ShapeDtypeStruct((2 * N, 128), jnp.float32),
        jax.ShapeDtypeStruct((N,), jnp.float32),
        jax.ShapeDtypeStruct((N,), jnp.int32),
    ),
    mesh=_mesh,
    scratch_types=[
        pltpu.VMEM((B_PER_W,), jnp.int32),
        pltpu.VMEM((2 * B_PER_W,), jnp.int32),
        pltpu.VMEM((2, HR, 128), jnp.float32),
        pltpu.VMEM((2, HR, 128), jnp.float32),
        pltpu.VMEM((B_PER_W,), jnp.float32),
        pltpu.SemaphoreType.DMA,
        pltpu.SemaphoreType.DMA,
        pltpu.SemaphoreType.DMA,
        pltpu.SemaphoreType.DMA,
        pltpu.SemaphoreType.DMA,
        pltpu.SemaphoreType.DMA,
    ],
    compiler_params=pltpu.CompilerParams(
        use_tc_tiling_on_sc=False, needs_layout_passes=False),
)
def _vq_sc(idx_hbm, lat_hbm, w_hbm, q_hbm, loss_hbm, idxo_hbm,
           idx_v, idx2_v, rows2, lat2, loss_v,
           sg0, sg1, sl0, sl1, so0, so1):
    wid = lax.axis_index("s") * NC + lax.axis_index("c")
    base = wid * B_PER_W
    rbase = 2 * base
    pltpu.sync_copy(idx_hbm.at[pl.ds(base, B_PER_W)], idx_v)
    pltpu.sync_copy(idx_v, idxo_hbm.at[pl.ds(base, B_PER_W)])
    scale = jnp.float32((1.0 + BETA) / D)
    lane = lax.iota(jnp.int32, L)
    masks = [(lane & d) == 0 for d in (1, 2, 4, 8)]

    def expand(g16, carry):
        k = idx_v[pl.ds(g16 * L, L)]
        i0 = ((k >> 3) << 4) | (k & 7)
        j = g16 * L + lane
        p0 = ((j >> 3) << 4) | (j & 7)
        plsc.store_scatter(idx2_v, [p0], i0)
        plsc.store_scatter(idx2_v, [p0 | 8], i0 | 8)
        return carry

    lax.fori_loop(0, B_PER_W // L, expand, 0)

    sgs = [sg0, sg1]
    sls = [sl0, sl1]
    sos = [so0, so1]

    def issue(c):
        b = c % 2
        g = pltpu.async_copy(
            w_hbm.at[idx2_v.at[pl.ds(c * HR, HR)]], rows2.at[b], sgs[b])
        l = pltpu.async_copy(
            lat_hbm.at[pl.ds(rbase + c * HR, HR)], lat2.at[b], sls[b])
        return g, l

    pend = issue(0)
    outs = [None, None]
    for c in range(NCHUNK):
        b = c % 2
        pend[0].wait()
        pend[1].wait()
        if c + 1 < NCHUNK:
            nb = (c + 1) % 2
            if outs[nb] is not None:
                outs[nb].wait()
                outs[nb] = None
            pend = issue(c + 1)
        outc = pltpu.async_copy(
            rows2.at[b], q_hbm.at[pl.ds(rbase + c * HR, HR)], sos[b])
        outs[b] = outc

        @plsc.parallel_loop(0, CHUNK // L)
        def group_body(g):
            accs = []
            for u in range(L):
                a0 = jnp.zeros((L,), jnp.float32)
                a1 = jnp.zeros((L,), jnp.float32)
                for cc in range(2):
                    r = 32 * g + 16 * (u >> 3) + (u & 7) + 8 * cc
                    for j in range(128 // L):
                        e = (rows2[b, r, pl.ds(j * L, L)]
                             - lat2[b, r, pl.ds(j * L, L)])
                        if j % 2 == 0:
                            a0 = a0 + e * e
                        else:
                            a1 = a1 + e * e
                accs.append(a0 + a1)
            for si, dist in enumerate((1, 2, 4, 8)):
                nxt = []
                for p in range(0, len(accs), 2):
                    x, y = accs[p], accs[p + 1]
                    px = x.at[lane ^ dist].get(mode="promise_in_bounds")
                    py = y.at[lane ^ dist].get(mode="promise_in_bounds")
                    nxt.append(jnp.where(masks[si], x + px, y + py))
                accs = nxt
            loss_v[pl.ds(c * CHUNK + g * L, L)] = accs[0] * scale
    for o in outs:
        if o is not None:
            o.wait()
    pltpu.sync_copy(loss_v, loss_hbm.at[pl.ds(base, B_PER_W)])


def kernel(gold_encoding_inds, latents, epc, W):
    bs, t, d = latents.shape
    n = bs * t
    idx = gold_encoding_inds[:, 0].astype(jnp.int32)
    w_r = W.reshape(K // 8, 8, 2, 128).transpose(0, 2, 1, 3).reshape(2 * K, 128)
    lat_r = latents.reshape(bs, t // 8, 8, 2, 128).transpose(0, 1, 3, 2, 4)
    lat_r = lat_r.reshape(2 * n, 128)
    q_r, loss, idx_out = _vq_sc(idx, lat_r, w_r)
    gold_quantized = (q_r.reshape(bs, t // 8, 2, 8, 128)
                      .transpose(0, 1, 3, 2, 4).reshape(bs, t, d))
    vq_loss = loss.reshape(bs, t)
    inds_t = idx_out.astype(gold_encoding_inds.dtype).reshape(1, n)
    return gold_quantized, vq_loss, inds_t
